# baseline (speedup 1.0000x reference)
import functools

import jax
import jax.numpy as jnp
from jax import lax
from jax.experimental import pallas as pl
from jax.experimental.pallas import tpu as pltpu
from jax.experimental.pallas import tpu_sc as plsc

_NC = 2
_NS = 16
_NW = _NC * _NS
_LANES = 16
_CHUNK = 128
_GROUP = 8
_DP = 8


@functools.lru_cache(maxsize=None)
def _make_sc_gather(F, V, per_w, n_chunks):
    mesh = plsc.VectorSubcoreMesh(core_axis_name="c", subcore_axis_name="s")

    @functools.partial(
        pl.kernel,
        out_type=jax.ShapeDtypeStruct((_NW, per_w, _DP), jnp.float32),
        mesh=mesh,
        scratch_types=[
            pltpu.VMEM((per_w,), jnp.int32),
            pltpu.VMEM((per_w, _DP), jnp.float32),
            pltpu.SemaphoreType.DMA,
        ],
        compiler_params=pltpu.CompilerParams(use_tc_tiling_on_sc=False),
    )
    def gather_k(comp_hbm, table_hbm, out_hbm, idx_v, rows_v, sem):
        wid = lax.axis_index("s") * _NC + lax.axis_index("c")
        base = wid * per_w
        pltpu.sync_copy(comp_hbm.at[wid], idx_v)
        iota = lax.iota(jnp.int32, _LANES)

        def group_body(g, carry):
            gbase = g * (_CHUNK * _GROUP)
            copies = []
            for k in range(_GROUP):
                cbase = gbase + k * _CHUNK
                for c in range(_CHUNK // _LANES):
                    o = cbase + c * _LANES
                    v = idx_v[pl.ds(o, _LANES)]
                    n = base + o + iota
                    idx_v[pl.ds(o, _LANES)] = v + (n % F) * V
                copies.append(pltpu.async_copy(
                    table_hbm.at[idx_v.at[pl.ds(cbase, _CHUNK)]],
                    rows_v.at[pl.ds(cbase, _CHUNK)],
                    sem,
                ))
            for cp in copies:
                cp.wait()
            return carry

        lax.fori_loop(0, n_chunks // _GROUP, group_body, 0)
        pltpu.sync_copy(rows_v, out_hbm.at[wid])

    return gather_k


def _ln(x, g, b, eps=1e-5):
    m = jnp.mean(x, axis=-1, keepdims=True)
    c = x - m
    v = jnp.mean(c * c, axis=-1, keepdims=True)
    return c * lax.rsqrt(v + eps) * g + b


def _make_mlp_body(F, d):
    def body(ep, meta, w1a, w1b, b1, g1, bb1, w2, b2, g2, bb2, w3, b3,
             emb_out, pred_out):
        hp = jax.lax.Precision.HIGHEST
        x = ep[...]
        emb_out[...] = jnp.concatenate(
            [x[:, i * _DP:i * _DP + d] for i in range(F)], axis=1)
        h = jnp.dot(x, w1a[...], precision=hp, preferred_element_type=jnp.float32)
        h = h + jnp.dot(meta[...], w1b[...], precision=hp,
                        preferred_element_type=jnp.float32)
        h = _ln(h + b1[...], g1[...], bb1[...])
        h = jnp.where(h >= 0, h, 0.01 * h)
        h = jnp.dot(h, w2[...], precision=hp, preferred_element_type=jnp.float32)
        h = _ln(h + b2[...], g2[...], bb2[...])
        h = jnp.where(h >= 0, h, 0.01 * h)
        pred_out[...] = jnp.dot(h, w3[...], precision=hp,
                                preferred_element_type=jnp.float32) + b3[...]
    return body


@functools.lru_cache(maxsize=None)
def _make_tc_mlp(B, F, d, meta_dim, d_model, BM):
    Kp = F * _DP
    full = lambda shape: pl.BlockSpec(shape, lambda i: (0, 0))
    return pl.pallas_call(
        _make_mlp_body(F, d),
        grid=(B // BM,),
        in_specs=[
            pl.BlockSpec((BM, Kp), lambda i: (i, 0)),
            pl.BlockSpec((BM, meta_dim), lambda i: (i, 0)),
            full((Kp, d_model)),
            full((meta_dim, d_model)),
            full((1, d_model)),
            full((1, d_model)),
            full((1, d_model)),
            full((d_model, d_model)),
            full((1, d_model)),
            full((1, d_model)),
            full((1, d_model)),
            full((d_model, 1)),
            full((1, 1)),
        ],
        out_specs=[
            pl.BlockSpec((BM, F * d), lambda i: (i, 0)),
            pl.BlockSpec((BM, 1), lambda i: (i, 0)),
        ],
        out_shape=[
            jax.ShapeDtypeStruct((B, F * d), jnp.float32),
            jax.ShapeDtypeStruct((B, 1), jnp.float32),
        ],
    )


def kernel(components, meta_feature, tables,
           W1, b1, ln1_g, ln1_b, W2, b2, ln2_g, ln2_b, W3, b3):
    B, F = components.shape
    V, d = tables.shape[1], tables.shape[2]
    meta_dim = meta_feature.shape[1]
    d_model = W1.shape[1]
    N = B * F
    per_w = N // _NW
    n_chunks = per_w // _CHUNK
    assert N == per_w * _NW and per_w == n_chunks * _CHUNK
    assert n_chunks % _GROUP == 0

    comp = components.astype(jnp.int32).reshape(_NW, per_w)
    table_pad = jnp.pad(tables.reshape(F * V, d), ((0, 0), (0, _DP - d)))
    emb_pad = _make_sc_gather(F, V, per_w, n_chunks)(comp, table_pad)
    emb_pad = emb_pad.reshape(B, F * _DP)

    W1a = jnp.pad(W1[:F * d].reshape(F, d, d_model),
                  ((0, 0), (0, _DP - d), (0, 0))).reshape(F * _DP, d_model)
    row = lambda a: a.reshape(1, -1)
    emb, pred = _make_tc_mlp(B, F, d, meta_dim, d_model, 512)(
        emb_pad, meta_feature,
        W1a, W1[F * d:], row(b1), row(ln1_g), row(ln1_b),
        W2, row(b2), row(ln2_g), row(ln2_b),
        W3, b3.reshape(1, 1))
    return (emb, pred)
